# Initial kernel scaffold; baseline (speedup 1.0000x reference)
import jax
import jax.numpy as jnp
from jax import lax
from jax.experimental import pallas as pl
from jax.experimental.pallas import tpu as pltpu
from jax.experimental.pallas import tpu_sc as plsc

N = 10000
E = 320000
D = 128
NC = 2
NS = 16
NW = NC * NS
EPT = E // NW
K = 80
NCHUNK = EPT // K
NPAD = 10240
EPS = NPAD // NS
RPS = N // NS


def _mesh():
    return plsc.VectorSubcoreMesh(core_axis_name="c", subcore_axis_name="s")


def _deg_kernel(dst_hbm, zero_hbm, degpart_hbm, dst_loc, ones_v, shared_deg):
    cid = lax.axis_index("c")
    sid = lax.axis_index("s")
    wid = sid * NC + cid
    pltpu.sync_copy(zero_hbm.at[pl.ds(sid * EPS, EPS)],
                    shared_deg.at[pl.ds(sid * EPS, EPS)])
    pltpu.sync_copy(dst_hbm.at[wid], dst_loc)
    for i in range(K // 16):
        ones_v[pl.ds(i * 16, 16)] = jnp.full((16,), 1.0, jnp.float32)
    plsc.subcore_barrier()

    def body(c, carry):
        pltpu.sync_copy(ones_v, shared_deg.at[dst_loc.at[c]], add=True)
        return carry

    lax.fori_loop(0, NCHUNK, body, 0)
    plsc.subcore_barrier()
    pltpu.sync_copy(shared_deg.at[pl.ds(sid * EPS, EPS)],
                    degpart_hbm.at[pl.ds(cid * NPAD + sid * EPS, EPS)])


def _deg_pass(dst3, zeros_pad):
    k = pl.kernel(
        _deg_kernel,
        out_type=jax.ShapeDtypeStruct((NC * NPAD,), jnp.float32),
        mesh=_mesh(),
        scratch_types=[
            pltpu.VMEM((NCHUNK, K), jnp.int32),
            pltpu.VMEM((K,), jnp.float32),
            pltpu.VMEM_SHARED((NPAD,), jnp.float32),
        ],
    )
    return k(dst3, zeros_pad)


def _edge_kernel(x_hbm, u_hbm, v_hbm, src_hbm, dst_hbm, zh_hbm, ze_hbm,
                 hpart_hbm, epart_hbm,
                 u_loc, v_loc, sidx, didx, rows, ex_buf, shared_h, shared_e,
                 sem):
    cid = lax.axis_index("c")
    sid = lax.axis_index("s")
    wid = sid * NC + cid
    pltpu.sync_copy(zh_hbm.at[pl.ds(sid * RPS, RPS)],
                    shared_h.at[pl.ds(sid * RPS, RPS)])
    pltpu.sync_copy(ze_hbm.at[pl.ds(sid * EPS, EPS)],
                    shared_e.at[pl.ds(sid * EPS, EPS)])
    pltpu.sync_copy(u_hbm, u_loc)
    pltpu.sync_copy(v_hbm, v_loc)
    pltpu.sync_copy(src_hbm.at[wid], sidx)
    pltpu.sync_copy(dst_hbm.at[wid], didx)
    plsc.subcore_barrier()

    def chunk(c, carry):
        gat = pltpu.async_copy(x_hbm.at[sidx.at[c]], rows, sem)
        for i in range(K // 16):
            si = sidx[c, pl.ds(i * 16, 16)]
            di = didx[c, pl.ds(i * 16, 16)]
            e = plsc.load_gather(u_loc, [si]) + plsc.load_gather(v_loc, [di])
            e = jnp.where(e >= 0.0, e, e * 0.2)
            ex_buf[pl.ds(i * 16, 16)] = jnp.exp(e)
        pltpu.sync_copy(ex_buf, shared_e.at[didx.at[c]], add=True)
        gat.wait()

        def scale(kk, inner):
            s = ex_buf[kk]
            for db in range(D // 16):
                rows[kk, pl.ds(db * 16, 16)] = rows[kk, pl.ds(db * 16, 16)] * s
            return inner

        lax.fori_loop(0, K, scale, 0)
        pltpu.sync_copy(rows, shared_h.at[didx.at[c]], add=True)
        return carry

    lax.fori_loop(0, NCHUNK, chunk, 0)
    plsc.subcore_barrier()
    pltpu.sync_copy(shared_h.at[pl.ds(sid * RPS, RPS)],
                    hpart_hbm.at[pl.ds(cid * N + sid * RPS, RPS)])
    pltpu.sync_copy(shared_e.at[pl.ds(sid * EPS, EPS)],
                    epart_hbm.at[pl.ds(cid * NPAD + sid * EPS, EPS)])


def _edge_pass(x, u, v, src3, dst3, zeros_h, zeros_pad):
    k = pl.kernel(
        _edge_kernel,
        out_type=(jax.ShapeDtypeStruct((NC * N, D), jnp.float32),
                  jax.ShapeDtypeStruct((NC * NPAD,), jnp.float32)),
        mesh=_mesh(),
        scratch_types=[
            pltpu.VMEM((N,), jnp.float32),
            pltpu.VMEM((N,), jnp.float32),
            pltpu.VMEM((NCHUNK, K), jnp.int32),
            pltpu.VMEM((NCHUNK, K), jnp.int32),
            pltpu.VMEM((K, D), jnp.float32),
            pltpu.VMEM((K,), jnp.float32),
            pltpu.VMEM_SHARED((N, D), jnp.float32),
            pltpu.VMEM_SHARED((NPAD,), jnp.float32),
            pltpu.SemaphoreType.DMA,
        ],
    )
    return k(x, u, v, src3, dst3, zeros_h, zeros_pad)


_GRID = 8
_RB = N // _GRID


def _prologue_kernel(degpair_ref, feats_ref, wc_ref, norm_ref, u_ref, v_ref):
    deg = degpair_ref[:, 0:1] + degpair_ref[:, 1:2]
    norm = lax.rsqrt(jnp.maximum(deg, 1.0))
    pq = jnp.dot(feats_ref[...], wc_ref[...], preferred_element_type=jnp.float32)
    norm_ref[...] = norm
    u_ref[...] = norm * pq[:, 0:1]
    v_ref[...] = norm * pq[:, 1:2]


def _prologue(degpair, feats, wcat):
    return pl.pallas_call(
        _prologue_kernel,
        grid=(_GRID,),
        in_specs=[
            pl.BlockSpec((_RB, 2), lambda i: (i, 0)),
            pl.BlockSpec((_RB, D), lambda i: (i, 0)),
            pl.BlockSpec((D, 2), lambda i: (0, 0)),
        ],
        out_specs=[
            pl.BlockSpec((_RB, 1), lambda i: (i, 0)),
            pl.BlockSpec((_RB, 1), lambda i: (i, 0)),
            pl.BlockSpec((_RB, 1), lambda i: (i, 0)),
        ],
        out_shape=[jax.ShapeDtypeStruct((N, 1), jnp.float32)] * 3,
    )(degpair, feats, wcat)


def _epilogue_kernel(hp_ref, ep_ref, y_ref, norm_ref, wc_ref, sc_ref,
                     x_ref, yo_ref, u_ref, v_ref):
    es = ep_ref[:, 0:1] + ep_ref[:, 1:2] + 1e-16
    h = (hp_ref[0] + hp_ref[1]) / es
    x_ref[...] = h
    yo_ref[...] = (y_ref[...] + h) * sc_ref[0, 0]
    norm = norm_ref[...]
    pq = jnp.dot(h, wc_ref[...], preferred_element_type=jnp.float32)
    u_ref[...] = norm * pq[:, 0:1]
    v_ref[...] = norm * pq[:, 1:2]


def _epilogue(hpart, epair, y_prev, norm, wcat, sc):
    return pl.pallas_call(
        _epilogue_kernel,
        grid=(_GRID,),
        in_specs=[
            pl.BlockSpec((NC, _RB, D), lambda i: (0, i, 0)),
            pl.BlockSpec((_RB, 2), lambda i: (i, 0)),
            pl.BlockSpec((_RB, D), lambda i: (i, 0)),
            pl.BlockSpec((_RB, 1), lambda i: (i, 0)),
            pl.BlockSpec((D, 2), lambda i: (0, 0)),
            pl.BlockSpec((1, 1), lambda i: (0, 0)),
        ],
        out_specs=[
            pl.BlockSpec((_RB, D), lambda i: (i, 0)),
            pl.BlockSpec((_RB, D), lambda i: (i, 0)),
            pl.BlockSpec((_RB, 1), lambda i: (i, 0)),
            pl.BlockSpec((_RB, 1), lambda i: (i, 0)),
        ],
        out_shape=[
            jax.ShapeDtypeStruct((N, D), jnp.float32),
            jax.ShapeDtypeStruct((N, D), jnp.float32),
            jax.ShapeDtypeStruct((N, 1), jnp.float32),
            jax.ShapeDtypeStruct((N, 1), jnp.float32),
        ],
    )(hpart, epair, y_prev, norm, wcat, sc)


def kernel(feats, edge_index, order, W_att):
    src3 = edge_index[0].astype(jnp.int32).reshape(NW, NCHUNK, K)
    dst3 = edge_index[1].astype(jnp.int32).reshape(NW, NCHUNK, K)
    wcat = W_att[:, 0].reshape(2, D).T
    zeros_pad = jnp.zeros((NPAD,), jnp.float32)
    zeros_h = jnp.zeros((N, D), jnp.float32)

    degpart = _deg_pass(dst3, zeros_pad)
    degpair = degpart.reshape(NC, NPAD)[:, :N].T
    norm, u, v = _prologue(degpair, feats, wcat)

    one = jnp.ones((1, 1), jnp.float32)
    last = (1.0 / (order + 1.0)) * one

    x = feats
    y = feats
    for t in range(4):
        hflat, eflat = _edge_pass(x, u.reshape(N), v.reshape(N),
                                  src3, dst3, zeros_h, zeros_pad)
        hpart = hflat.reshape(NC, N, D)
        epair = eflat.reshape(NC, NPAD)[:, :N].T
        sc = last if t == 3 else one
        x, y, u, v = _epilogue(hpart, epair, y, norm, wcat, sc)
    return y


# speedup vs baseline: 15.0599x; 15.0599x over previous
import jax
import jax.numpy as jnp
from jax import lax
from jax.experimental import pallas as pl
from jax.experimental.pallas import tpu as pltpu
from jax.experimental.pallas import tpu_sc as plsc

N = 10000
E = 320000
D = 128
NC = 2
NS = 16
NW = NC * NS
EPT = E // NW
K = 80
NCHUNK = EPT // K
NPAD = 10240
EPS = NPAD // NS
RPS = N // NS


def _mesh():
    return plsc.VectorSubcoreMesh(core_axis_name="c", subcore_axis_name="s")


def _deg_kernel(dst_hbm, zero_hbm, degpart_hbm, dst_loc, ones_v, shared_deg):
    cid = lax.axis_index("c")
    sid = lax.axis_index("s")
    wid = sid * NC + cid
    pltpu.sync_copy(zero_hbm.at[pl.ds(sid * EPS, EPS)],
                    shared_deg.at[pl.ds(sid * EPS, EPS)])
    pltpu.sync_copy(dst_hbm.at[wid], dst_loc)
    for i in range(K // 16):
        ones_v[pl.ds(i * 16, 16)] = jnp.full((16,), 1.0, jnp.float32)
    plsc.subcore_barrier()

    def body(c, carry):
        pltpu.sync_copy(ones_v, shared_deg.at[dst_loc.at[c]], add=True)
        return carry

    lax.fori_loop(0, NCHUNK, body, 0)
    plsc.subcore_barrier()
    pltpu.sync_copy(shared_deg.at[pl.ds(sid * EPS, EPS)],
                    degpart_hbm.at[pl.ds(cid * NPAD + sid * EPS, EPS)])


def _deg_pass(dst3, zeros_pad):
    k = pl.kernel(
        _deg_kernel,
        out_type=jax.ShapeDtypeStruct((NC * NPAD,), jnp.float32),
        mesh=_mesh(),
        compiler_params=pltpu.CompilerParams(needs_layout_passes=False),
        scratch_types=[
            pltpu.VMEM((NCHUNK, K), jnp.int32),
            pltpu.VMEM((K,), jnp.float32),
            pltpu.VMEM_SHARED((NPAD,), jnp.float32),
        ],
    )
    return k(dst3, zeros_pad)


def _edge_kernel(x_hbm, u_hbm, v_hbm, src_hbm, dst_hbm, zh_hbm, ze_hbm,
                 hpart_hbm, epart_hbm,
                 u_loc, v_loc, sidx, didx, rows, ex_buf, shared_h, shared_e,
                 sem):
    cid = lax.axis_index("c")
    sid = lax.axis_index("s")
    wid = sid * NC + cid
    pltpu.sync_copy(zh_hbm.at[pl.ds(sid * EPS, EPS)],
                    shared_h.at[pl.ds(sid * EPS, EPS)])
    pltpu.sync_copy(ze_hbm.at[pl.ds(sid * EPS, EPS)],
                    shared_e.at[pl.ds(sid * EPS, EPS)])
    pltpu.sync_copy(u_hbm, u_loc)
    pltpu.sync_copy(v_hbm, v_loc)
    plsc.subcore_barrier()

    def chunk(c, carry):
        pltpu.sync_copy(src_hbm.at[wid, c], sidx)
        pltpu.sync_copy(dst_hbm.at[wid, c], didx)
        gat = pltpu.async_copy(x_hbm.at[sidx], rows, sem)
        for i in range(K // 16):
            si = sidx[pl.ds(i * 16, 16)]
            di = didx[pl.ds(i * 16, 16)]
            e = plsc.load_gather(u_loc, [si]) + plsc.load_gather(v_loc, [di])
            e = jnp.where(e >= 0.0, e, e * 0.2)
            ex_buf[pl.ds(i * 16, 16)] = jnp.exp(e)
        pltpu.sync_copy(ex_buf, shared_e.at[didx], add=True)
        gat.wait()

        def scale(kk, inner):
            s = plsc.load_gather(ex_buf, [jnp.full((16,), 0, jnp.int32) + kk])
            for db in range(D // 16):
                rows[kk, pl.ds(db * 16, 16)] = rows[kk, pl.ds(db * 16, 16)] * s
            return inner

        lax.fori_loop(0, K, scale, 0)
        pltpu.sync_copy(rows, shared_h.at[didx], add=True)
        return carry

    lax.fori_loop(0, NCHUNK, chunk, 0)
    plsc.subcore_barrier()
    pltpu.sync_copy(shared_h.at[pl.ds(sid * EPS, EPS)],
                    hpart_hbm.at[pl.ds(cid * NPAD + sid * EPS, EPS)])
    pltpu.sync_copy(shared_e.at[pl.ds(sid * EPS, EPS)],
                    epart_hbm.at[pl.ds(cid * NPAD + sid * EPS, EPS)])


def _edge_pass(x, u, v, src3, dst3, zeros_h, zeros_pad):
    k = pl.kernel(
        _edge_kernel,
        out_type=(jax.ShapeDtypeStruct((NC * NPAD, D), jnp.float32),
                  jax.ShapeDtypeStruct((NC * NPAD,), jnp.float32)),
        mesh=_mesh(),
        compiler_params=pltpu.CompilerParams(needs_layout_passes=False),
        scratch_types=[
            pltpu.VMEM((N,), jnp.float32),
            pltpu.VMEM((N,), jnp.float32),
            pltpu.VMEM((K,), jnp.int32),
            pltpu.VMEM((K,), jnp.int32),
            pltpu.VMEM((K, D), jnp.float32),
            pltpu.VMEM((K,), jnp.float32),
            pltpu.VMEM_SHARED((NPAD, D), jnp.float32),
            pltpu.VMEM_SHARED((NPAD,), jnp.float32),
            pltpu.SemaphoreType.DMA,
        ],
    )
    return k(x, u, v, src3, dst3, zeros_h, zeros_pad)


_GRID = 10
_RB = N // _GRID


def _prologue_kernel(degpair_ref, feats_ref, wc_ref, norm_ref, u_ref, v_ref):
    deg = degpair_ref[:, 0:1] + degpair_ref[:, 1:2]
    norm = lax.rsqrt(jnp.maximum(deg, 1.0))
    pq = jnp.dot(feats_ref[...], wc_ref[...], preferred_element_type=jnp.float32)
    norm_ref[...] = norm
    u_ref[...] = norm * pq[:, 0:1]
    v_ref[...] = norm * pq[:, 1:2]


def _prologue(degpair, feats, wcat):
    return pl.pallas_call(
        _prologue_kernel,
        grid=(_GRID,),
        in_specs=[
            pl.BlockSpec((_RB, 2), lambda i: (i, 0)),
            pl.BlockSpec((_RB, D), lambda i: (i, 0)),
            pl.BlockSpec((D, 2), lambda i: (0, 0)),
        ],
        out_specs=[
            pl.BlockSpec((_RB, 1), lambda i: (i, 0)),
            pl.BlockSpec((_RB, 1), lambda i: (i, 0)),
            pl.BlockSpec((_RB, 1), lambda i: (i, 0)),
        ],
        out_shape=[jax.ShapeDtypeStruct((N, 1), jnp.float32)] * 3,
    )(degpair, feats, wcat)


def _epilogue_kernel(hp_ref, ep_ref, y_ref, norm_ref, wc_ref, sc_ref,
                     x_ref, yo_ref, u_ref, v_ref):
    es = ep_ref[:, 0:1] + ep_ref[:, 1:2] + 1e-16
    h = (hp_ref[0] + hp_ref[1]) / es
    x_ref[...] = h
    yo_ref[...] = (y_ref[...] + h) * sc_ref[0, 0]
    norm = norm_ref[...]
    pq = jnp.dot(h, wc_ref[...], preferred_element_type=jnp.float32)
    u_ref[...] = norm * pq[:, 0:1]
    v_ref[...] = norm * pq[:, 1:2]


def _epilogue(hpart, epair, y_prev, norm, wcat, sc):
    return pl.pallas_call(
        _epilogue_kernel,
        grid=(_GRID,),
        in_specs=[
            pl.BlockSpec((NC, _RB, D), lambda i: (0, i, 0)),
            pl.BlockSpec((_RB, 2), lambda i: (i, 0)),
            pl.BlockSpec((_RB, D), lambda i: (i, 0)),
            pl.BlockSpec((_RB, 1), lambda i: (i, 0)),
            pl.BlockSpec((D, 2), lambda i: (0, 0)),
            pl.BlockSpec((1, 1), lambda i: (0, 0)),
        ],
        out_specs=[
            pl.BlockSpec((_RB, D), lambda i: (i, 0)),
            pl.BlockSpec((_RB, D), lambda i: (i, 0)),
            pl.BlockSpec((_RB, 1), lambda i: (i, 0)),
            pl.BlockSpec((_RB, 1), lambda i: (i, 0)),
        ],
        out_shape=[
            jax.ShapeDtypeStruct((N, D), jnp.float32),
            jax.ShapeDtypeStruct((N, D), jnp.float32),
            jax.ShapeDtypeStruct((N, 1), jnp.float32),
            jax.ShapeDtypeStruct((N, 1), jnp.float32),
        ],
    )(hpart, epair, y_prev, norm, wcat, sc)


def kernel(feats, edge_index, order, W_att):
    src3 = edge_index[0].astype(jnp.int32).reshape(NW, NCHUNK, K)
    dst3 = edge_index[1].astype(jnp.int32).reshape(NW, NCHUNK, K)
    wcat = W_att[:, 0].reshape(2, D).T
    zeros_pad = jnp.zeros((NPAD,), jnp.float32)
    zeros_h = jnp.zeros((NPAD, D), jnp.float32)

    degpart = _deg_pass(dst3, zeros_pad)
    degpair = degpart.reshape(NC, NPAD)[:, :N].T
    norm, u, v = _prologue(degpair, feats, wcat)

    one = jnp.ones((1, 1), jnp.float32)
    last = (1.0 / (order + 1.0)) * one

    x = feats
    y = feats
    for t in range(4):
        hflat, eflat = _edge_pass(x, u.reshape(N), v.reshape(N),
                                  src3, dst3, zeros_h, zeros_pad)
        hpart = hflat.reshape(NC, NPAD, D)
        epair = eflat.reshape(NC, NPAD)[:, :N].T
        sc = last if t == 3 else one
        x, y, u, v = _epilogue(hpart, epair, y, norm, wcat, sc)
    return y


# speedup vs baseline: 21.5699x; 1.4323x over previous
import jax
import jax.numpy as jnp
from jax import lax
from jax.experimental import pallas as pl
from jax.experimental.pallas import tpu as pltpu
from jax.experimental.pallas import tpu_sc as plsc

N = 10000
E = 320000
D = 128
NC = 2
NS = 16
NW = NC * NS
EPT = E // NW
K = 80
NCHUNK = EPT // K
NPAD = 10240
EPS = NPAD // NS
RPS = N // NS


def _mesh():
    return plsc.VectorSubcoreMesh(core_axis_name="c", subcore_axis_name="s")


def _deg_kernel(sd_hbm, zero_hbm, degpart_hbm, sdl, ones_v, shared_deg):
    cid = lax.axis_index("c")
    sid = lax.axis_index("s")
    wid = sid * NC + cid
    pltpu.sync_copy(zero_hbm.at[pl.ds(sid * EPS, EPS)],
                    shared_deg.at[pl.ds(sid * EPS, EPS)])
    pltpu.sync_copy(sd_hbm.at[wid], sdl)
    for i in range(K // 16):
        ones_v[pl.ds(i * 16, 16)] = jnp.full((16,), 1.0, jnp.float32)
    plsc.subcore_barrier()

    def body(c, carry):
        pltpu.sync_copy(ones_v, shared_deg.at[sdl.at[c, 1]], add=True)
        return carry

    lax.fori_loop(0, NCHUNK, body, 0)
    plsc.subcore_barrier()
    pltpu.sync_copy(shared_deg.at[pl.ds(sid * EPS, EPS)],
                    degpart_hbm.at[pl.ds(cid * NPAD + sid * EPS, EPS)])


def _deg_pass(sd, zeros_pad):
    k = pl.kernel(
        _deg_kernel,
        out_type=jax.ShapeDtypeStruct((NC * NPAD,), jnp.float32),
        mesh=_mesh(),
        compiler_params=pltpu.CompilerParams(needs_layout_passes=False),
        scratch_types=[
            pltpu.VMEM((NCHUNK, 2, K), jnp.int32),
            pltpu.VMEM((K,), jnp.float32),
            pltpu.VMEM_SHARED((NPAD,), jnp.float32),
        ],
    )
    return k(sd, zeros_pad)


PAIRS = (NCHUNK - 1) // 2


def _edge_kernel(x_hbm, u_hbm, v_hbm, sd_hbm, zh_hbm, ze_hbm,
                 hpart_hbm, epart_hbm,
                 sdbuf, rows2, ex2, scal, shared_h, shared_e,
                 sem_ra, sem_rb, sem_ua, sem_va, sem_ub, sem_vb,
                 sem_ea, sem_eb, sem_ha, sem_hb):
    cid = lax.axis_index("c")
    sid = lax.axis_index("s")
    wid = sid * NC + cid
    pltpu.sync_copy(zh_hbm.at[pl.ds(sid * EPS, EPS)],
                    shared_h.at[pl.ds(sid * EPS, EPS)])
    pltpu.sync_copy(ze_hbm.at[pl.ds(sid * EPS, EPS)],
                    shared_e.at[pl.ds(sid * EPS, EPS)])

    z16 = jnp.zeros((16,), jnp.float32)

    def zr(j, carry):
        for db in range(D // 16):
            rows2[0, j, pl.ds(db * 16, 16)] = z16
            rows2[1, j, pl.ds(db * 16, 16)] = z16
        return carry

    lax.fori_loop(0, K, zr, 0)
    for i in range(K // 16):
        ex2[0, pl.ds(i * 16, 16)] = z16
        ex2[1, pl.ds(i * 16, 16)] = z16
    plsc.subcore_barrier()

    pltpu.sync_copy(sd_hbm.at[wid, pl.ds(0, 2)], sdbuf)
    pltpu.async_copy(ex2.at[0], shared_e.at[sdbuf.at[0, 1]], sem_ea, add=True)
    pltpu.async_copy(ex2.at[1], shared_e.at[sdbuf.at[1, 1]], sem_eb, add=True)
    pltpu.async_copy(rows2.at[0], shared_h.at[sdbuf.at[0, 1]], sem_ha, add=True)
    pltpu.async_copy(rows2.at[1], shared_h.at[sdbuf.at[1, 1]], sem_hb, add=True)

    def process_slot(slot, du, dv, dr, se, sh):
        du.wait()
        dv.wait()
        for i in range(K // 16):
            e = scal[slot, 0, pl.ds(i * 16, 16)] + scal[slot, 1, pl.ds(i * 16, 16)]
            e = jnp.where(e >= 0.0, e, e * 0.2)
            ex2[slot, pl.ds(i * 16, 16)] = jnp.exp(e)
        pltpu.async_copy(ex2.at[slot], shared_e.at[sdbuf.at[slot, 1]], se,
                         add=True)
        dr.wait()

        def scale(kk, inner):
            s = plsc.load_gather(ex2.at[slot],
                                 [jnp.zeros((16,), jnp.int32) + kk])
            for db in range(D // 16):
                rows2[slot, kk, pl.ds(db * 16, 16)] = (
                    rows2[slot, kk, pl.ds(db * 16, 16)] * s)
            return inner

        lax.fori_loop(0, K, scale, 0)
        pltpu.async_copy(rows2.at[slot], shared_h.at[sdbuf.at[slot, 1]], sh,
                         add=True)

    def drain_all():
        pltpu.make_async_copy(ex2.at[0], shared_e.at[sdbuf.at[0, 1]], sem_ea).wait()
        pltpu.make_async_copy(ex2.at[1], shared_e.at[sdbuf.at[1, 1]], sem_eb).wait()
        pltpu.make_async_copy(rows2.at[0], shared_h.at[sdbuf.at[0, 1]], sem_ha).wait()
        pltpu.make_async_copy(rows2.at[1], shared_h.at[sdbuf.at[1, 1]], sem_hb).wait()

    def body(i, carry):
        drain_all()
        pltpu.sync_copy(sd_hbm.at[wid, pl.ds(2 * i, 2)], sdbuf)
        dra = pltpu.async_copy(x_hbm.at[sdbuf.at[0, 0]], rows2.at[0], sem_ra)
        drb = pltpu.async_copy(x_hbm.at[sdbuf.at[1, 0]], rows2.at[1], sem_rb)
        dua = pltpu.async_copy(u_hbm.at[sdbuf.at[0, 0]], scal.at[0, 0], sem_ua)
        dva = pltpu.async_copy(v_hbm.at[sdbuf.at[0, 1]], scal.at[0, 1], sem_va)
        dub = pltpu.async_copy(u_hbm.at[sdbuf.at[1, 0]], scal.at[1, 0], sem_ub)
        dvb = pltpu.async_copy(v_hbm.at[sdbuf.at[1, 1]], scal.at[1, 1], sem_vb)
        process_slot(0, dua, dva, dra, sem_ea, sem_ha)
        process_slot(1, dub, dvb, drb, sem_eb, sem_hb)
        return carry

    lax.fori_loop(0, PAIRS, body, 0)
    drain_all()
    pltpu.sync_copy(sd_hbm.at[wid, NCHUNK - 1], sdbuf.at[0])
    dra = pltpu.async_copy(x_hbm.at[sdbuf.at[0, 0]], rows2.at[0], sem_ra)
    dua = pltpu.async_copy(u_hbm.at[sdbuf.at[0, 0]], scal.at[0, 0], sem_ua)
    dva = pltpu.async_copy(v_hbm.at[sdbuf.at[0, 1]], scal.at[0, 1], sem_va)
    process_slot(0, dua, dva, dra, sem_ea, sem_ha)
    pltpu.make_async_copy(ex2.at[0], shared_e.at[sdbuf.at[0, 1]], sem_ea).wait()
    pltpu.make_async_copy(rows2.at[0], shared_h.at[sdbuf.at[0, 1]], sem_ha).wait()
    plsc.subcore_barrier()
    pltpu.sync_copy(shared_h.at[pl.ds(sid * EPS, EPS)],
                    hpart_hbm.at[pl.ds(cid * NPAD + sid * EPS, EPS)])
    pltpu.sync_copy(shared_e.at[pl.ds(sid * EPS, EPS)],
                    epart_hbm.at[pl.ds(cid * NPAD + sid * EPS, EPS)])


def _edge_pass(x, u, v, sd, zeros_h, zeros_pad):
    k = pl.kernel(
        _edge_kernel,
        out_type=(jax.ShapeDtypeStruct((NC * NPAD, D), jnp.float32),
                  jax.ShapeDtypeStruct((NC * NPAD,), jnp.float32)),
        mesh=_mesh(),
        compiler_params=pltpu.CompilerParams(needs_layout_passes=False),
        scratch_types=[
            pltpu.VMEM((2, 2, K), jnp.int32),
            pltpu.VMEM((2, K, D), jnp.float32),
            pltpu.VMEM((2, K), jnp.float32),
            pltpu.VMEM((2, 2, K), jnp.float32),
            pltpu.VMEM_SHARED((NPAD, D), jnp.float32),
            pltpu.VMEM_SHARED((NPAD,), jnp.float32),
        ] + [pltpu.SemaphoreType.DMA] * 10,
    )
    return k(x, u, v, sd, zeros_h, zeros_pad)


_GRID = 10
_RB = N // _GRID


def _prologue_kernel(degpair_ref, feats_ref, wc_ref, norm_ref, u_ref, v_ref):
    deg = degpair_ref[:, 0:1] + degpair_ref[:, 1:2]
    norm = lax.rsqrt(jnp.maximum(deg, 1.0))
    pq = jnp.dot(feats_ref[...], wc_ref[...], preferred_element_type=jnp.float32)
    norm_ref[...] = norm
    u_ref[...] = norm * pq[:, 0:1]
    v_ref[...] = norm * pq[:, 1:2]


def _prologue(degpair, feats, wcat):
    return pl.pallas_call(
        _prologue_kernel,
        grid=(_GRID,),
        in_specs=[
            pl.BlockSpec((_RB, 2), lambda i: (i, 0)),
            pl.BlockSpec((_RB, D), lambda i: (i, 0)),
            pl.BlockSpec((D, 2), lambda i: (0, 0)),
        ],
        out_specs=[
            pl.BlockSpec((_RB, 1), lambda i: (i, 0)),
            pl.BlockSpec((_RB, 1), lambda i: (i, 0)),
            pl.BlockSpec((_RB, 1), lambda i: (i, 0)),
        ],
        out_shape=[jax.ShapeDtypeStruct((N, 1), jnp.float32)] * 3,
    )(degpair, feats, wcat)


def _epilogue_kernel(hp_ref, ep_ref, y_ref, norm_ref, wc_ref, sc_ref,
                     x_ref, yo_ref, u_ref, v_ref):
    es = ep_ref[:, 0:1] + ep_ref[:, 1:2] + 1e-16
    h = (hp_ref[0] + hp_ref[1]) / es
    x_ref[...] = h
    yo_ref[...] = (y_ref[...] + h) * sc_ref[0, 0]
    norm = norm_ref[...]
    pq = jnp.dot(h, wc_ref[...], preferred_element_type=jnp.float32)
    u_ref[...] = norm * pq[:, 0:1]
    v_ref[...] = norm * pq[:, 1:2]


def _epilogue(hpart, epair, y_prev, norm, wcat, sc):
    return pl.pallas_call(
        _epilogue_kernel,
        grid=(_GRID,),
        in_specs=[
            pl.BlockSpec((NC, _RB, D), lambda i: (0, i, 0)),
            pl.BlockSpec((_RB, 2), lambda i: (i, 0)),
            pl.BlockSpec((_RB, D), lambda i: (i, 0)),
            pl.BlockSpec((_RB, 1), lambda i: (i, 0)),
            pl.BlockSpec((D, 2), lambda i: (0, 0)),
            pl.BlockSpec((1, 1), lambda i: (0, 0)),
        ],
        out_specs=[
            pl.BlockSpec((_RB, D), lambda i: (i, 0)),
            pl.BlockSpec((_RB, D), lambda i: (i, 0)),
            pl.BlockSpec((_RB, 1), lambda i: (i, 0)),
            pl.BlockSpec((_RB, 1), lambda i: (i, 0)),
        ],
        out_shape=[
            jax.ShapeDtypeStruct((N, D), jnp.float32),
            jax.ShapeDtypeStruct((N, D), jnp.float32),
            jax.ShapeDtypeStruct((N, 1), jnp.float32),
            jax.ShapeDtypeStruct((N, 1), jnp.float32),
        ],
    )(hpart, epair, y_prev, norm, wcat, sc)


def kernel(feats, edge_index, order, W_att):
    src3 = edge_index[0].astype(jnp.int32).reshape(NW, NCHUNK, K)
    dst3 = edge_index[1].astype(jnp.int32).reshape(NW, NCHUNK, K)
    sd = jnp.stack([src3, dst3], axis=2)
    wcat = W_att[:, 0].reshape(2, D).T
    zeros_pad = jnp.zeros((NPAD,), jnp.float32)
    zeros_h = jnp.zeros((NPAD, D), jnp.float32)

    degpart = _deg_pass(sd, zeros_pad)
    degpair = degpart.reshape(NC, NPAD)[:, :N].T
    norm, u, v = _prologue(degpair, feats, wcat)

    one = jnp.ones((1, 1), jnp.float32)
    last = (1.0 / (order + 1.0)) * one

    x = feats
    y = feats
    for t in range(4):
        hflat, eflat = _edge_pass(x, u.reshape(N), v.reshape(N),
                                  sd, zeros_h, zeros_pad)
        hpart = hflat.reshape(NC, NPAD, D)
        epair = eflat.reshape(NC, NPAD)[:, :N].T
        sc = last if t == 3 else one
        x, y, u, v = _epilogue(hpart, epair, y, norm, wcat, sc)
    return y
